# baseline (speedup 1.0000x reference)
import functools

import jax
import jax.numpy as jnp
from jax import lax
from jax.experimental import pallas as pl
from jax.experimental.pallas import tpu as pltpu
from jax.experimental.pallas import tpu_sc as plsc

VOCAB = 100002
D = 128
B = 4
L = 4096
Y = 50
FILTER_SIZES = [3, 5, 9]
CONV_DIMS = [128, 100, 50]
NFM = 50
LATENT = len(FILTER_SIZES) * NFM // 2
FEAT = len(FILTER_SIZES) * NFM
CP = 128
FEATP = len(FILTER_SIZES) * CP

SC_NC = 2
SC_NS = 16
SC_NW = SC_NC * SC_NS


def _sc_gather(table, idx_flat):
    n = idx_flat.shape[0]
    b_per_w = n // SC_NW
    ch = 128
    nch = b_per_w // ch
    idx2 = idx_flat.reshape(SC_NW * nch, ch)
    mesh = plsc.VectorSubcoreMesh(core_axis_name="c", subcore_axis_name="s")

    @functools.partial(
        pl.kernel,
        mesh=mesh,
        out_type=jax.ShapeDtypeStruct((n, D), jnp.float32),
        scratch_types=[
            pltpu.VMEM((nch, ch), jnp.int32),
            pltpu.VMEM((b_per_w, D), jnp.float32),
            pltpu.SemaphoreType.DMA,
        ],
    )
    def gk(table_hbm, idx_hbm, out_hbm, idx_v, rows_v, sem):
        wid = lax.axis_index("s") * SC_NC + lax.axis_index("c")
        pltpu.sync_copy(idx_hbm.at[pl.ds(wid * nch, nch)], idx_v)
        copies = [
            pltpu.async_copy(
                table_hbm.at[idx_v.at[j]], rows_v.at[pl.ds(j * ch, ch)], sem
            )
            for j in range(nch)
        ]
        for c in copies:
            c.wait()
        pltpu.sync_copy(rows_v, out_hbm.at[pl.ds(wid * b_per_w, b_per_w)])

    return gk(table, idx2)


def _fold_bn(w, g, b, m, v):
    s = g / jnp.sqrt(v + 1e-5)
    return w * s[:, None, None], b - m * s


def _prep_conv(w, bias):
    cout, cin, k = w.shape
    wt = jnp.transpose(w, (2, 1, 0))
    wt = jnp.pad(wt, ((0, 0), (0, CP - cin), (0, CP - cout)))
    bp = jnp.pad(bias, (0, CP - cout)).reshape(1, CP)
    return wt.reshape(k * CP, CP).astype(jnp.bfloat16), bp


PADROW = 8


def _stage(tp_ref, t):
    tp_ref[pl.ds(PADROW, L), :] = t


def _conv(tp_ref, w_ref, b, k):
    pad = k // 2
    parts = [
        tp_ref[pl.ds(PADROW + dk - pad, L), :].astype(jnp.bfloat16)
        for dk in range(k)
    ]
    xcat = jnp.concatenate(parts, axis=1)
    return jnp.dot(xcat, w_ref[...], preferred_element_type=jnp.float32) + b


def _tc_kernel_body(refs, *, nweights):
    (emb_ref, tgt_ref, eps_ref), wrefs, (y_ref, bce_ref, kl_ref), tp_ref = (
        refs[:3], refs[3:3 + nweights], refs[3 + nweights:-1], refs[-1])
    wi = iter(wrefs)

    def nxt():
        return next(wi)

    tp_ref[pl.ds(0, PADROW), :] = jnp.zeros((PADROW, CP), jnp.float32)
    tp_ref[pl.ds(PADROW + L, PADROW), :] = jnp.zeros((PADROW, CP), jnp.float32)

    res = []
    for k in FILTER_SIZES:
        w0, b0 = nxt(), nxt()[...]
        _stage(tp_ref, emb_ref[0])
        t = jnp.tanh(_conv(tp_ref, w0, b0, k))
        for _blk in range(2):
            w1, b1 = nxt(), nxt()[...]
            w2, b2 = nxt(), nxt()[...]
            ws, bs = nxt()[...], nxt()[...]
            _stage(tp_ref, t)
            h1 = jnp.tanh(_conv(tp_ref, w1, b1, k))
            _stage(tp_ref, h1)
            h2 = _conv(tp_ref, w2, b2, k)
            sc = jnp.dot(t.astype(jnp.bfloat16), ws,
                         preferred_element_type=jnp.float32) + bs
            t = jnp.tanh(h2 + sc)
        res.append(t.astype(jnp.bfloat16))
    xc = jnp.concatenate(res, axis=1)

    uwt = nxt()[...]
    muw = nxt()[...]
    mub = nxt()[...]
    vaw = nxt()[...]
    vab = nxt()[...]
    fw = nxt()[...]
    fb = nxt()[...]

    xcb = xc
    scores = jnp.dot(xcb, uwt, preferred_element_type=jnp.float32)
    smax = jnp.max(scores, axis=0, keepdims=True)
    e = jnp.exp(scores - smax)
    alpha = e / jnp.sum(e, axis=0, keepdims=True)
    m = lax.dot_general(
        alpha.astype(jnp.bfloat16), xcb, (((0,), (0,)), ((), ())),
        preferred_element_type=jnp.float32,
    )

    mu = jnp.sum(muw * m, axis=1, keepdims=True) + mub
    lv = jnp.sum(vaw * m, axis=1, keepdims=True) + vab
    epsv = jnp.transpose(eps_ref[0])
    z = epsv * jnp.exp(0.5 * lv) + mu
    yv = jnp.sum(fw * jnp.transpose(z), axis=1, keepdims=True) + fb
    yt = jnp.transpose(yv)
    y_ref[0] = yt

    tgt = tgt_ref[0]
    bce_el = jnp.maximum(yt, 0.0) - yt * tgt + jnp.log1p(jnp.exp(-jnp.abs(yt)))
    bce_ref[0] = jnp.sum(bce_el).reshape(1, 1)
    kl_ref[0] = (-0.5 * jnp.sum(1.0 + lv - mu * mu - jnp.exp(lv))).reshape(1, 1)


def _prep_weights(params):
    ws = []
    for ch in params["channels"]:
        w0, b0 = _prep_conv(ch["base_w"], ch["base_b"])
        ws += [w0, b0]
        for blk in ch["blocks"]:
            w1, bb1 = _fold_bn(blk["w1"], blk["bn1_g"], blk["bn1_b"],
                               blk["bn1_m"], blk["bn1_v"])
            w2, bb2 = _fold_bn(blk["w2"], blk["bn2_g"], blk["bn2_b"],
                               blk["bn2_m"], blk["bn2_v"])
            wsc, bbs = _fold_bn(blk["ws"], blk["bns_g"], blk["bns_b"],
                                blk["bns_m"], blk["bns_v"])
            p1, pb1 = _prep_conv(w1, bb1)
            p2, pb2 = _prep_conv(w2, bb2)
            ps, pbs = _prep_conv(wsc, bbs)
            ws += [p1, pb1, p2, pb2, ps, pbs]
    nc = len(FILTER_SIZES)

    def padf(a):
        out = jnp.zeros((a.shape[0], FEATP), a.dtype)
        for c in range(nc):
            out = out.at[:, c * CP:c * CP + NFM].set(a[:, c * NFM:(c + 1) * NFM])
        return out

    ws.append(jnp.transpose(padf(params["U_w"])).astype(jnp.bfloat16))
    ws.append(padf(params["mu_w"]))
    ws.append(params["mu_b"].reshape(LATENT, 1))
    ws.append(padf(params["var_w"]))
    ws.append(params["var_b"].reshape(LATENT, 1))
    ws.append(params["final_w"])
    ws.append(params["final_b"].reshape(Y, 1))
    return ws


def _tc_forward(emb, target, eps, weights):
    nw = len(weights)

    def const_spec(a):
        return pl.BlockSpec(a.shape, lambda b: (0,) * a.ndim)

    in_specs = [
        pl.BlockSpec((1, L, D), lambda b: (b, 0, 0)),
        pl.BlockSpec((1, 1, Y), lambda b: (b, 0, 0)),
        pl.BlockSpec((1, 1, LATENT), lambda b: (b, 0, 0)),
    ] + [const_spec(w) for w in weights]
    out_specs = [
        pl.BlockSpec((1, 1, Y), lambda b: (b, 0, 0)),
        pl.BlockSpec((1, 1, 1), lambda b: (b, 0, 0)),
        pl.BlockSpec((1, 1, 1), lambda b: (b, 0, 0)),
    ]
    out_shape = [
        jax.ShapeDtypeStruct((B, 1, Y), jnp.float32),
        jax.ShapeDtypeStruct((B, 1, 1), jnp.float32),
        jax.ShapeDtypeStruct((B, 1, 1), jnp.float32),
    ]

    def body(*refs):
        _tc_kernel_body(refs, nweights=nw)

    y3, bce3, kl3 = pl.pallas_call(
        body,
        grid=(B,),
        in_specs=in_specs,
        out_specs=out_specs,
        out_shape=out_shape,
        scratch_shapes=[pltpu.VMEM((L + 2 * PADROW, CP), jnp.float32)],
        compiler_params=pltpu.CompilerParams(
            dimension_semantics=("parallel",)),
    )(emb.reshape(B, L, D), target.reshape(B, 1, Y),
      eps.reshape(B, 1, LATENT), *weights)
    return y3, bce3, kl3


def kernel(x, target, text_inputs, eps, params):
    del text_inputs
    emb = _sc_gather(params["embed"], x.reshape(-1))
    weights = _prep_weights(params)
    y3, bce3, kl3 = _tc_forward(emb, target, eps, weights)
    y = y3.reshape(B, Y)
    bce = jnp.sum(bce3) / (B * Y)
    kl = jnp.sum(kl3) / B
    return y, bce, kl


# speedup vs baseline: 1.0077x; 1.0077x over previous
import functools

import jax
import jax.numpy as jnp
from jax import lax
from jax.experimental import pallas as pl
from jax.experimental.pallas import tpu as pltpu
from jax.experimental.pallas import tpu_sc as plsc

VOCAB = 100002
D = 128
B = 4
L = 4096
Y = 50
FILTER_SIZES = [3, 5, 9]
CONV_DIMS = [128, 100, 50]
NFM = 50
LATENT = len(FILTER_SIZES) * NFM // 2
FEAT = len(FILTER_SIZES) * NFM
CP = 128
NCH = len(FILTER_SIZES)
FEATP = NCH * CP

_CH_OFF = []
_off = 0
for _k in FILTER_SIZES:
    _CH_OFF.append(_off)
    _off += 5 * _k * CP
_SC_OFF = _off
WROWS = _SC_OFF + 2 * NCH * CP

SC_NC = 2
SC_NS = 16
SC_NW = SC_NC * SC_NS


def _sc_gather(table, idx_flat):
    n = idx_flat.shape[0]
    b_per_w = n // SC_NW
    ch = 128
    nch = b_per_w // ch
    idx2 = idx_flat.reshape(SC_NW * nch, ch)
    mesh = plsc.VectorSubcoreMesh(core_axis_name="c", subcore_axis_name="s")

    @functools.partial(
        pl.kernel,
        mesh=mesh,
        out_type=jax.ShapeDtypeStruct((n, D), jnp.float32),
        scratch_types=[
            pltpu.VMEM((nch, ch), jnp.int32),
            pltpu.VMEM((b_per_w, D), jnp.float32),
            pltpu.SemaphoreType.DMA,
        ],
    )
    def gk(table_hbm, idx_hbm, out_hbm, idx_v, rows_v, sem):
        wid = lax.axis_index("s") * SC_NC + lax.axis_index("c")
        pltpu.sync_copy(idx_hbm.at[pl.ds(wid * nch, nch)], idx_v)
        copies = [
            pltpu.async_copy(
                table_hbm.at[idx_v.at[j]], rows_v.at[pl.ds(j * ch, ch)], sem
            )
            for j in range(nch)
        ]
        for c in copies:
            c.wait()
        pltpu.sync_copy(rows_v, out_hbm.at[pl.ds(wid * b_per_w, b_per_w)])

    return gk(table, idx2)


def _folded(blk, wname, pre):
    w = blk[wname]
    s = blk[pre + "_g"] / jnp.sqrt(blk[pre + "_v"] + 1e-5)
    return w * s[:, None, None], blk[pre + "_b"] - blk[pre + "_m"] * s


def _pad3(w):
    cout, cin, k = w.shape
    return jnp.pad(w, ((0, CP - cout), (0, CP - cin), (0, 0)))


def _padb(b):
    return jnp.pad(b, (0, CP - b.shape[0]))


def _padf(a):
    z = jnp.zeros((a.shape[0], CP - NFM), a.dtype)
    cols = []
    for c in range(NCH):
        cols += [a[:, c * NFM:(c + 1) * NFM], z]
    return jnp.concatenate(cols, axis=1)


def _prep_weights(params):
    wparts, brows, scs = [], [], []
    for ch in params["channels"]:
        k = ch["base_w"].shape[2]
        taps = [_pad3(ch["base_w"])]
        brows.append(_padb(ch["base_b"]))
        for blk in ch["blocks"]:
            w1, b1 = _folded(blk, "w1", "bn1")
            w2, b2 = _folded(blk, "w2", "bn2")
            wsc, bsc = _folded(blk, "ws", "bns")
            taps += [_pad3(w1), _pad3(w2)]
            scs.append(_pad3(wsc)[:, :, 0])
            brows += [_padb(b1), _padb(b2), _padb(bsc)]
        stack = jnp.stack(taps)
        stack = jnp.transpose(stack, (0, 3, 2, 1))
        wparts.append(stack.reshape(5 * k * CP, CP))
    scstack = jnp.transpose(jnp.stack(scs), (0, 2, 1))
    wparts.append(scstack.reshape(2 * NCH * CP, CP))
    wconv = jnp.concatenate(wparts, axis=0)
    bbuf = jnp.stack(brows)

    uwt = jnp.transpose(_padf(params["U_w"]))
    mvw = jnp.concatenate([
        _padf(params["mu_w"]), jnp.zeros((5, FEATP), jnp.float32),
        _padf(params["var_w"]), jnp.zeros((5, FEATP), jnp.float32),
    ], axis=0)
    sv = jnp.stack([params["mu_b"], params["var_b"]], axis=1)
    fw = params["final_w"]
    fb = params["final_b"].reshape(Y, 1)
    return [wconv, bbuf, uwt, mvw, sv, fw, fb]


def _shiftv(x, off):
    if off == 0:
        return x
    z = jnp.zeros((abs(off), x.shape[1]), x.dtype)
    if off > 0:
        return jnp.concatenate([x[off:], z], axis=0)
    return jnp.concatenate([z, x[:off]], axis=0)


def _conv(x, wc_ref, row, b, k):
    pad = k // 2
    xcat = jnp.concatenate([_shiftv(x, dk - pad) for dk in range(k)], axis=1)
    w = wc_ref[pl.ds(row, k * CP), :]
    return jnp.dot(xcat, w, preferred_element_type=jnp.float32) + b


def _tc_kernel_body(emb_ref, tgt_ref, eps_ref, wc_ref, bb_ref, uwt_ref,
                    mvw_ref, sv_ref, fw_ref, fb_ref, y_ref, bce_ref, kl_ref):
    x = emb_ref[0]

    res = []
    for ci, k in enumerate(FILTER_SIZES):
        row = _CH_OFF[ci]
        bias = lambda j: bb_ref[pl.ds(7 * ci + j, 1), :]
        t = jnp.tanh(_conv(x, wc_ref, row, bias(0), k))
        for blk in range(2):
            r1 = row + (1 + 2 * blk) * k * CP
            r2 = row + (2 + 2 * blk) * k * CP
            rs = _SC_OFF + (2 * ci + blk) * CP
            h1 = jnp.tanh(_conv(t, wc_ref, r1, bias(1 + 3 * blk), k))
            h2 = _conv(h1, wc_ref, r2, bias(2 + 3 * blk), k)
            ws = wc_ref[pl.ds(rs, CP), :]
            sc = jnp.dot(t, ws, preferred_element_type=jnp.float32)
            sc = sc + bias(3 + 3 * blk)
            t = jnp.tanh(h2 + sc)
        res.append(t)
    xc = jnp.concatenate(res, axis=1)

    scores = jnp.dot(xc, uwt_ref[...],
                     preferred_element_type=jnp.float32)
    smax = jnp.max(scores, axis=0, keepdims=True)
    e = jnp.exp(scores - smax)
    alpha = e / jnp.sum(e, axis=0, keepdims=True)
    m = lax.dot_general(
        alpha, xc, (((0,), (0,)), ((), ())),
        preferred_element_type=jnp.float32,
    )

    muw = mvw_ref[pl.ds(0, LATENT), :]
    vaw = mvw_ref[pl.ds(80, LATENT), :]
    mu = jnp.sum(muw * m, axis=1, keepdims=True) + sv_ref[:, 0:1]
    lv = jnp.sum(vaw * m, axis=1, keepdims=True) + sv_ref[:, 1:2]
    epsv = jnp.transpose(eps_ref[0])
    z = epsv * jnp.exp(0.5 * lv) + mu
    yv = jnp.sum(fw_ref[...] * jnp.transpose(z), axis=1,
                 keepdims=True) + fb_ref[...]
    yt = jnp.transpose(yv)
    y_ref[0] = yt

    tgt = tgt_ref[0]
    bce_el = jnp.maximum(yt, 0.0) - yt * tgt + jnp.log1p(jnp.exp(-jnp.abs(yt)))
    bce_ref[0] = jnp.sum(bce_el).reshape(1, 1)
    kl_ref[0] = (-0.5 * jnp.sum(1.0 + lv - mu * mu - jnp.exp(lv))).reshape(1, 1)


def _tc_forward(emb, target, eps, weights):
    def const_spec(a):
        return pl.BlockSpec(a.shape, lambda b: (0,) * a.ndim)

    in_specs = [
        pl.BlockSpec((1, L, D), lambda b: (b, 0, 0)),
        pl.BlockSpec((1, 1, Y), lambda b: (b, 0, 0)),
        pl.BlockSpec((1, 1, LATENT), lambda b: (b, 0, 0)),
    ] + [const_spec(w) for w in weights]
    out_specs = [
        pl.BlockSpec((1, 1, Y), lambda b: (b, 0, 0)),
        pl.BlockSpec((1, 1, 1), lambda b: (b, 0, 0)),
        pl.BlockSpec((1, 1, 1), lambda b: (b, 0, 0)),
    ]
    out_shape = [
        jax.ShapeDtypeStruct((B, 1, Y), jnp.float32),
        jax.ShapeDtypeStruct((B, 1, 1), jnp.float32),
        jax.ShapeDtypeStruct((B, 1, 1), jnp.float32),
    ]

    y3, bce3, kl3 = pl.pallas_call(
        _tc_kernel_body,
        grid=(B,),
        in_specs=in_specs,
        out_specs=out_specs,
        out_shape=out_shape,
        compiler_params=pltpu.CompilerParams(
            dimension_semantics=("arbitrary",)),
    )(emb.reshape(B, L, D), target.reshape(B, 1, Y),
      eps.reshape(B, 1, LATENT), *weights)
    return y3, bce3, kl3


def kernel(x, target, text_inputs, eps, params):
    del text_inputs
    emb = _sc_gather(params["embed"], x.reshape(-1))
    weights = _prep_weights(params)
    y3, bce3, kl3 = _tc_forward(emb, target, eps, weights)
    y = y3.reshape(B, Y)
    bce = jnp.sum(bce3) / (B * Y)
    kl = jnp.sum(kl3) / B
    return y, bce, kl


# speedup vs baseline: 1.1141x; 1.1056x over previous
import functools

import jax
import jax.numpy as jnp
from jax import lax
from jax.experimental import pallas as pl
from jax.experimental.pallas import tpu as pltpu
from jax.experimental.pallas import tpu_sc as plsc

VOCAB = 100002
D = 128
B = 4
L = 4096
Y = 50
FILTER_SIZES = [3, 5, 9]
CONV_DIMS = [128, 100, 50]
NFM = 50
LATENT = len(FILTER_SIZES) * NFM // 2
FEAT = len(FILTER_SIZES) * NFM
CP = 128
NCH = len(FILTER_SIZES)
FEATP = NCH * CP

_CH_OFF = []
_off = 0
for _k in FILTER_SIZES:
    _CH_OFF.append(_off)
    _off += 5 * _k * CP
_SC_OFF = _off
WROWS = _SC_OFF + 2 * NCH * CP

SC_NC = 2
SC_NS = 16
SC_NW = SC_NC * SC_NS


def _sc_gather(table, idx_flat):
    n = idx_flat.shape[0]
    b_per_w = n // SC_NW
    ch = 128
    nch = b_per_w // ch
    idx2 = idx_flat.reshape(SC_NW * nch, ch)
    mesh = plsc.VectorSubcoreMesh(core_axis_name="c", subcore_axis_name="s")

    @functools.partial(
        pl.kernel,
        mesh=mesh,
        out_type=jax.ShapeDtypeStruct((n, D), jnp.float32),
        scratch_types=[
            pltpu.VMEM((nch, ch), jnp.int32),
            pltpu.VMEM((b_per_w, D), jnp.float32),
            pltpu.SemaphoreType.DMA,
        ],
    )
    def gk(table_hbm, idx_hbm, out_hbm, idx_v, rows_v, sem):
        wid = lax.axis_index("s") * SC_NC + lax.axis_index("c")
        pltpu.sync_copy(idx_hbm.at[pl.ds(wid * nch, nch)], idx_v)
        copies = [
            pltpu.async_copy(
                table_hbm.at[idx_v.at[j]], rows_v.at[pl.ds(j * ch, ch)], sem
            )
            for j in range(nch)
        ]
        for c in copies:
            c.wait()
        pltpu.sync_copy(rows_v, out_hbm.at[pl.ds(wid * b_per_w, b_per_w)])

    return gk(table, idx2)


def _folded(blk, wname, pre):
    w = blk[wname]
    s = blk[pre + "_g"] / jnp.sqrt(blk[pre + "_v"] + 1e-5)
    return w * s[:, None, None], blk[pre + "_b"] - blk[pre + "_m"] * s


def _pad3(w):
    cout, cin, k = w.shape
    return jnp.pad(w, ((0, CP - cout), (0, CP - cin), (0, 0)))


def _padb(b):
    return jnp.pad(b, (0, CP - b.shape[0]))


def _padf(a):
    z = jnp.zeros((a.shape[0], CP - NFM), a.dtype)
    cols = []
    for c in range(NCH):
        cols += [a[:, c * NFM:(c + 1) * NFM], z]
    return jnp.concatenate(cols, axis=1)


def _prep_weights(params):
    wparts, brows, scs = [], [], []
    for ch in params["channels"]:
        k = ch["base_w"].shape[2]
        taps = [_pad3(ch["base_w"])]
        brows.append(_padb(ch["base_b"]))
        for blk in ch["blocks"]:
            w1, b1 = _folded(blk, "w1", "bn1")
            w2, b2 = _folded(blk, "w2", "bn2")
            wsc, bsc = _folded(blk, "ws", "bns")
            taps += [_pad3(w1), _pad3(w2)]
            scs.append(_pad3(wsc)[:, :, 0])
            brows += [_padb(b1), _padb(b2), _padb(bsc)]
        stack = jnp.stack(taps)
        stack = jnp.transpose(stack, (0, 3, 2, 1))
        wparts.append(stack.reshape(5 * k * CP, CP))
    scstack = jnp.transpose(jnp.stack(scs), (0, 2, 1))
    wparts.append(scstack.reshape(2 * NCH * CP, CP))
    wconv = jnp.concatenate(wparts, axis=0)
    bbuf = jnp.stack(brows)

    uwt = jnp.transpose(_padf(params["U_w"]))
    mvw = jnp.concatenate([
        _padf(params["mu_w"]), jnp.zeros((5, FEATP), jnp.float32),
        _padf(params["var_w"]), jnp.zeros((5, FEATP), jnp.float32),
    ], axis=0)
    sv = jnp.stack([params["mu_b"], params["var_b"]], axis=1)
    fw = params["final_w"]
    fb = params["final_b"].reshape(Y, 1)
    return [wconv, bbuf, uwt, mvw, sv, fw, fb]


def _shiftv(x, off):
    if off == 0:
        return x
    z = jnp.zeros((abs(off), x.shape[1]), x.dtype)
    if off > 0:
        return jnp.concatenate([x[off:], z], axis=0)
    return jnp.concatenate([z, x[:off]], axis=0)


def _conv(x, wc_ref, row, b, k):
    pad = k // 2
    acc = jnp.dot(x, wc_ref[pl.ds(row + pad * CP, CP), :],
                  preferred_element_type=jnp.float32)
    for dk in range(k):
        if dk == pad:
            continue
        y = jnp.dot(x, wc_ref[pl.ds(row + dk * CP, CP), :],
                    preferred_element_type=jnp.float32)
        acc = acc + _shiftv(y, dk - pad)
    return acc + b


def _tc_kernel_body(emb_ref, tgt_ref, eps_ref, wc_ref, bb_ref, uwt_ref,
                    mvw_ref, sv_ref, fw_ref, fb_ref, y_ref, bce_ref, kl_ref):
    x = emb_ref[0]

    res = []
    for ci, k in enumerate(FILTER_SIZES):
        row = _CH_OFF[ci]
        bias = lambda j: bb_ref[pl.ds(7 * ci + j, 1), :]
        t = jnp.tanh(_conv(x, wc_ref, row, bias(0), k))
        for blk in range(2):
            r1 = row + (1 + 2 * blk) * k * CP
            r2 = row + (2 + 2 * blk) * k * CP
            rs = _SC_OFF + (2 * ci + blk) * CP
            h1 = jnp.tanh(_conv(t, wc_ref, r1, bias(1 + 3 * blk), k))
            h2 = _conv(h1, wc_ref, r2, bias(2 + 3 * blk), k)
            ws = wc_ref[pl.ds(rs, CP), :]
            sc = jnp.dot(t, ws, preferred_element_type=jnp.float32)
            sc = sc + bias(3 + 3 * blk)
            t = jnp.tanh(h2 + sc)
        res.append(t)
    xc = jnp.concatenate(res, axis=1)

    scores = jnp.dot(xc, uwt_ref[...],
                     preferred_element_type=jnp.float32)
    smax = jnp.max(scores, axis=0, keepdims=True)
    e = jnp.exp(scores - smax)
    alpha = e / jnp.sum(e, axis=0, keepdims=True)
    m = lax.dot_general(
        alpha, xc, (((0,), (0,)), ((), ())),
        preferred_element_type=jnp.float32,
    )

    muw = mvw_ref[pl.ds(0, LATENT), :]
    vaw = mvw_ref[pl.ds(80, LATENT), :]
    mu = jnp.sum(muw * m, axis=1, keepdims=True) + sv_ref[:, 0:1]
    lv = jnp.sum(vaw * m, axis=1, keepdims=True) + sv_ref[:, 1:2]
    epsv = jnp.transpose(eps_ref[0])
    z = epsv * jnp.exp(0.5 * lv) + mu
    yv = jnp.sum(fw_ref[...] * jnp.transpose(z), axis=1,
                 keepdims=True) + fb_ref[...]
    yt = jnp.transpose(yv)
    y_ref[0] = yt

    tgt = tgt_ref[0]
    bce_el = jnp.maximum(yt, 0.0) - yt * tgt + jnp.log1p(jnp.exp(-jnp.abs(yt)))
    bce_ref[0] = jnp.sum(bce_el).reshape(1, 1)
    kl_ref[0] = (-0.5 * jnp.sum(1.0 + lv - mu * mu - jnp.exp(lv))).reshape(1, 1)


def _tc_forward(emb, target, eps, weights):
    def const_spec(a):
        return pl.BlockSpec(a.shape, lambda b: (0,) * a.ndim)

    in_specs = [
        pl.BlockSpec((1, L, D), lambda b: (b, 0, 0)),
        pl.BlockSpec((1, 1, Y), lambda b: (b, 0, 0)),
        pl.BlockSpec((1, 1, LATENT), lambda b: (b, 0, 0)),
    ] + [const_spec(w) for w in weights]
    out_specs = [
        pl.BlockSpec((1, 1, Y), lambda b: (b, 0, 0)),
        pl.BlockSpec((1, 1, 1), lambda b: (b, 0, 0)),
        pl.BlockSpec((1, 1, 1), lambda b: (b, 0, 0)),
    ]
    out_shape = [
        jax.ShapeDtypeStruct((B, 1, Y), jnp.float32),
        jax.ShapeDtypeStruct((B, 1, 1), jnp.float32),
        jax.ShapeDtypeStruct((B, 1, 1), jnp.float32),
    ]

    y3, bce3, kl3 = pl.pallas_call(
        _tc_kernel_body,
        grid=(B,),
        in_specs=in_specs,
        out_specs=out_specs,
        out_shape=out_shape,
        compiler_params=pltpu.CompilerParams(
            dimension_semantics=("arbitrary",)),
    )(emb.reshape(B, L, D), target.reshape(B, 1, Y),
      eps.reshape(B, 1, LATENT), *weights)
    return y3, bce3, kl3


def kernel(x, target, text_inputs, eps, params):
    del text_inputs
    emb = _sc_gather(params["embed"], x.reshape(-1))
    weights = _prep_weights(params)
    y3, bce3, kl3 = _tc_forward(emb, target, eps, weights)
    y = y3.reshape(B, Y)
    bce = jnp.sum(bce3) / (B * Y)
    kl = jnp.sum(kl3) / B
    return y, bce, kl


# speedup vs baseline: 1.2178x; 1.0932x over previous
import functools

import jax
import jax.numpy as jnp
from jax import lax
from jax.experimental import pallas as pl
from jax.experimental.pallas import tpu as pltpu
from jax.experimental.pallas import tpu_sc as plsc

VOCAB = 100002
D = 128
B = 4
L = 4096
Y = 50
FILTER_SIZES = [3, 5, 9]
CONV_DIMS = [128, 100, 50]
NFM = 50
LATENT = len(FILTER_SIZES) * NFM // 2
FEAT = len(FILTER_SIZES) * NFM
CP = 128
NCH = len(FILTER_SIZES)
FEATP = NCH * CP

_CH_OFF = []
_off = 0
for _k in FILTER_SIZES:
    _CH_OFF.append(_off)
    _off += 5 * _k * CP
_SC_OFF = _off
WROWS = _SC_OFF + 2 * NCH * CP

SC_NC = 2
SC_NS = 16
SC_NW = SC_NC * SC_NS


def _sc_gather(table, idx_flat):
    n = idx_flat.shape[0]
    b_per_w = n // SC_NW
    ch = 128
    nch = b_per_w // ch
    idx2 = idx_flat.reshape(SC_NW * nch, ch)
    mesh = plsc.VectorSubcoreMesh(core_axis_name="c", subcore_axis_name="s")

    @functools.partial(
        pl.kernel,
        mesh=mesh,
        out_type=jax.ShapeDtypeStruct((n, D), jnp.float32),
        scratch_types=[
            pltpu.VMEM((nch, ch), jnp.int32),
            pltpu.VMEM((b_per_w, D), jnp.float32),
            pltpu.SemaphoreType.DMA,
        ],
    )
    def gk(table_hbm, idx_hbm, out_hbm, idx_v, rows_v, sem):
        wid = lax.axis_index("s") * SC_NC + lax.axis_index("c")
        pltpu.sync_copy(idx_hbm.at[pl.ds(wid * nch, nch)], idx_v)
        copies = [
            pltpu.async_copy(
                table_hbm.at[idx_v.at[j]], rows_v.at[pl.ds(j * ch, ch)], sem
            )
            for j in range(nch)
        ]
        for c in copies:
            c.wait()
        pltpu.sync_copy(rows_v, out_hbm.at[pl.ds(wid * b_per_w, b_per_w)])

    return gk(table, idx2)


def _folded(blk, wname, pre):
    w = blk[wname]
    s = blk[pre + "_g"] / jnp.sqrt(blk[pre + "_v"] + 1e-5)
    return w * s[:, None, None], blk[pre + "_b"] - blk[pre + "_m"] * s


def _pad3(w):
    cout, cin, k = w.shape
    return jnp.pad(w, ((0, CP - cout), (0, CP - cin), (0, 0)))


def _padb(b):
    return jnp.pad(b, (0, CP - b.shape[0]))


def _padf(a):
    z = jnp.zeros((a.shape[0], CP - NFM), a.dtype)
    cols = []
    for c in range(NCH):
        cols += [a[:, c * NFM:(c + 1) * NFM], z]
    return jnp.concatenate(cols, axis=1)


def _prep_weights(params):
    wparts, brows, scs = [], [], []
    for ch in params["channels"]:
        k = ch["base_w"].shape[2]
        taps = [_pad3(ch["base_w"])]
        brows.append(_padb(ch["base_b"]))
        for blk in ch["blocks"]:
            w1, b1 = _folded(blk, "w1", "bn1")
            w2, b2 = _folded(blk, "w2", "bn2")
            wsc, bsc = _folded(blk, "ws", "bns")
            taps += [_pad3(w1), _pad3(w2)]
            scs.append(_pad3(wsc)[:, :, 0])
            brows += [_padb(b1), _padb(b2), _padb(bsc)]
        stack = jnp.stack(taps)
        stack = jnp.transpose(stack, (0, 3, 2, 1))
        wparts.append(stack.reshape(5 * k * CP, CP))
    scstack = jnp.transpose(jnp.stack(scs), (0, 2, 1))
    wparts.append(scstack.reshape(2 * NCH * CP, CP))
    wconv = jnp.concatenate(wparts, axis=0).astype(jnp.bfloat16)
    bbuf = jnp.stack(brows)

    uwt = jnp.transpose(_padf(params["U_w"])).astype(jnp.bfloat16)
    mvw = jnp.concatenate([
        _padf(params["mu_w"]), jnp.zeros((5, FEATP), jnp.float32),
        _padf(params["var_w"]), jnp.zeros((5, FEATP), jnp.float32),
    ], axis=0)
    sv = jnp.stack([params["mu_b"], params["var_b"]], axis=1)
    fw = params["final_w"]
    fb = params["final_b"].reshape(Y, 1)
    return [wconv, bbuf, uwt, mvw, sv, fw, fb]


def _shiftv(x, off):
    if off == 0:
        return x
    z = jnp.zeros((abs(off), x.shape[1]), x.dtype)
    if off > 0:
        return jnp.concatenate([x[off:], z], axis=0)
    return jnp.concatenate([z, x[:off]], axis=0)


def _conv(x, wc_ref, row, b, k):
    pad = k // 2
    xb = x.astype(jnp.bfloat16)
    acc = jnp.dot(xb, wc_ref[pl.ds(row + pad * CP, CP), :],
                  preferred_element_type=jnp.float32)
    for dk in range(k):
        if dk == pad:
            continue
        y = jnp.dot(xb, wc_ref[pl.ds(row + dk * CP, CP), :],
                    preferred_element_type=jnp.float32)
        acc = acc + _shiftv(y, dk - pad)
    return acc + b


def _tc_kernel_body(emb_ref, tgt_ref, eps_ref, wc_ref, bb_ref, uwt_ref,
                    mvw_ref, sv_ref, fw_ref, fb_ref, y_ref, bce_ref, kl_ref):
    x = emb_ref[0]

    res = []
    for ci, k in enumerate(FILTER_SIZES):
        row = _CH_OFF[ci]
        bias = lambda j: bb_ref[pl.ds(7 * ci + j, 1), :]
        t = jnp.tanh(_conv(x, wc_ref, row, bias(0), k))
        for blk in range(2):
            r1 = row + (1 + 2 * blk) * k * CP
            r2 = row + (2 + 2 * blk) * k * CP
            rs = _SC_OFF + (2 * ci + blk) * CP
            h1 = jnp.tanh(_conv(t, wc_ref, r1, bias(1 + 3 * blk), k))
            h2 = _conv(h1, wc_ref, r2, bias(2 + 3 * blk), k)
            ws = wc_ref[pl.ds(rs, CP), :]
            sc = jnp.dot(t.astype(jnp.bfloat16), ws,
                         preferred_element_type=jnp.float32)
            sc = sc + bias(3 + 3 * blk)
            t = jnp.tanh(h2 + sc)
        res.append(t)
    xc = jnp.concatenate(res, axis=1)

    xcb = xc.astype(jnp.bfloat16)
    scores = jnp.dot(xcb, uwt_ref[...],
                     preferred_element_type=jnp.float32)
    smax = jnp.max(scores, axis=0, keepdims=True)
    e = jnp.exp(scores - smax)
    alpha = e / jnp.sum(e, axis=0, keepdims=True)
    m = lax.dot_general(
        alpha.astype(jnp.bfloat16), xcb, (((0,), (0,)), ((), ())),
        preferred_element_type=jnp.float32,
    )

    muw = mvw_ref[pl.ds(0, LATENT), :]
    vaw = mvw_ref[pl.ds(80, LATENT), :]
    mu = jnp.sum(muw * m, axis=1, keepdims=True) + sv_ref[:, 0:1]
    lv = jnp.sum(vaw * m, axis=1, keepdims=True) + sv_ref[:, 1:2]
    epsv = jnp.transpose(eps_ref[0])
    z = epsv * jnp.exp(0.5 * lv) + mu
    yv = jnp.sum(fw_ref[...] * jnp.transpose(z), axis=1,
                 keepdims=True) + fb_ref[...]
    yt = jnp.transpose(yv)
    y_ref[0] = yt

    tgt = tgt_ref[0]
    bce_el = jnp.maximum(yt, 0.0) - yt * tgt + jnp.log1p(jnp.exp(-jnp.abs(yt)))
    bce_ref[0] = jnp.sum(bce_el).reshape(1, 1)
    kl_ref[0] = (-0.5 * jnp.sum(1.0 + lv - mu * mu - jnp.exp(lv))).reshape(1, 1)


def _tc_forward(emb, target, eps, weights):
    def const_spec(a):
        return pl.BlockSpec(a.shape, lambda b: (0,) * a.ndim)

    in_specs = [
        pl.BlockSpec((1, L, D), lambda b: (b, 0, 0)),
        pl.BlockSpec((1, 1, Y), lambda b: (b, 0, 0)),
        pl.BlockSpec((1, 1, LATENT), lambda b: (b, 0, 0)),
    ] + [const_spec(w) for w in weights]
    out_specs = [
        pl.BlockSpec((1, 1, Y), lambda b: (b, 0, 0)),
        pl.BlockSpec((1, 1, 1), lambda b: (b, 0, 0)),
        pl.BlockSpec((1, 1, 1), lambda b: (b, 0, 0)),
    ]
    out_shape = [
        jax.ShapeDtypeStruct((B, 1, Y), jnp.float32),
        jax.ShapeDtypeStruct((B, 1, 1), jnp.float32),
        jax.ShapeDtypeStruct((B, 1, 1), jnp.float32),
    ]

    y3, bce3, kl3 = pl.pallas_call(
        _tc_kernel_body,
        grid=(B,),
        in_specs=in_specs,
        out_specs=out_specs,
        out_shape=out_shape,
        compiler_params=pltpu.CompilerParams(
            dimension_semantics=("arbitrary",)),
    )(emb.reshape(B, L, D), target.reshape(B, 1, Y),
      eps.reshape(B, 1, LATENT), *weights)
    return y3, bce3, kl3


def kernel(x, target, text_inputs, eps, params):
    del text_inputs
    emb = _sc_gather(params["embed"], x.reshape(-1))
    weights = _prep_weights(params)
    y3, bce3, kl3 = _tc_forward(emb, target, eps, weights)
    y = y3.reshape(B, Y)
    bce = jnp.sum(bce3) / (B * Y)
    kl = jnp.sum(kl3) / B
    return y, bce, kl


# speedup vs baseline: 1.2941x; 1.0626x over previous
import functools

import jax
import jax.numpy as jnp
from jax import lax
from jax.experimental import pallas as pl
from jax.experimental.pallas import tpu as pltpu
from jax.experimental.pallas import tpu_sc as plsc

VOCAB = 100002
D = 128
B = 4
L = 4096
Y = 50
FILTER_SIZES = [3, 5, 9]
CONV_DIMS = [128, 100, 50]
NFM = 50
LATENT = len(FILTER_SIZES) * NFM // 2
FEAT = len(FILTER_SIZES) * NFM
CP = 128
FEATP = len(FILTER_SIZES) * CP

SC_NC = 2
SC_NS = 16
SC_NW = SC_NC * SC_NS


def _sc_gather(table, idx_flat):
    n = idx_flat.shape[0]
    b_per_w = n // SC_NW
    ch = 128
    nch = b_per_w // ch
    idx2 = idx_flat.reshape(SC_NW * nch, ch)
    mesh = plsc.VectorSubcoreMesh(core_axis_name="c", subcore_axis_name="s")

    @functools.partial(
        pl.kernel,
        mesh=mesh,
        out_type=jax.ShapeDtypeStruct((n, D), jnp.float32),
        scratch_types=[
            pltpu.VMEM((nch, ch), jnp.int32),
            pltpu.VMEM((b_per_w, D), jnp.float32),
            pltpu.SemaphoreType.DMA,
        ],
    )
    def gk(table_hbm, idx_hbm, out_hbm, idx_v, rows_v, sem):
        wid = lax.axis_index("s") * SC_NC + lax.axis_index("c")
        pltpu.sync_copy(idx_hbm.at[pl.ds(wid * nch, nch)], idx_v)
        copies = [
            pltpu.async_copy(
                table_hbm.at[idx_v.at[j]], rows_v.at[pl.ds(j * ch, ch)], sem
            )
            for j in range(nch)
        ]
        for c in copies:
            c.wait()
        pltpu.sync_copy(rows_v, out_hbm.at[pl.ds(wid * b_per_w, b_per_w)])

    return gk(table, idx2)


def _fold_bn(w, g, b, m, v):
    s = g / jnp.sqrt(v + 1e-5)
    return w * s[:, None, None], b - m * s


def _prep_conv(w, bias):
    cout, cin, k = w.shape
    wt = jnp.transpose(w, (2, 1, 0))
    wt = jnp.pad(wt, ((0, 0), (0, CP - cin), (0, CP - cout)))
    bp = jnp.pad(bias, (0, CP - cout)).reshape(1, CP)
    return wt.astype(jnp.bfloat16), bp


def _conv(x, w_ref, b, k):
    pad = k // 2
    xb = x.astype(jnp.bfloat16)
    acc = jnp.dot(xb, w_ref[pad], preferred_element_type=jnp.float32)
    for dk in range(k):
        if dk == pad:
            continue
        y = jnp.dot(xb, w_ref[dk], preferred_element_type=jnp.float32)
        off = dk - pad
        if off > 0:
            ysh = jnp.concatenate(
                [y[off:], jnp.zeros((off, y.shape[1]), y.dtype)], axis=0
            )
        else:
            ysh = jnp.concatenate(
                [jnp.zeros((-off, y.shape[1]), y.dtype), y[:off]], axis=0
            )
        acc = acc + ysh
    return acc + b


def _tc_kernel_body(refs, *, nweights):
    (emb_ref, tgt_ref, eps_ref), wrefs, (y_ref, bce_ref, kl_ref) = (
        refs[:3], refs[3:3 + nweights], refs[3 + nweights:])
    wi = iter(wrefs)

    def nxt():
        return next(wi)

    x = emb_ref[0]
    res = []
    for k in FILTER_SIZES:
        w0, b0 = nxt(), nxt()[...]
        t = jnp.tanh(_conv(x, w0, b0, k))
        for _blk in range(2):
            w1, b1 = nxt(), nxt()[...]
            w2, b2 = nxt(), nxt()[...]
            ws, bs = nxt()[...], nxt()[...]
            h1 = jnp.tanh(_conv(t, w1, b1, k))
            h2 = _conv(h1, w2, b2, k)
            sc = jnp.dot(t.astype(jnp.bfloat16), ws,
                         preferred_element_type=jnp.float32) + bs
            t = jnp.tanh(h2 + sc)
        res.append(t)
    xc = jnp.concatenate(res, axis=1)

    uwt = nxt()[...]
    muw = nxt()[...]
    mub = nxt()[...]
    vaw = nxt()[...]
    vab = nxt()[...]
    fw = nxt()[...]
    fb = nxt()[...]

    xcb = xc.astype(jnp.bfloat16)
    scores = jnp.dot(xcb, uwt, preferred_element_type=jnp.float32)
    smax = jnp.max(scores, axis=0, keepdims=True)
    e = jnp.exp(scores - smax)
    alpha = e / jnp.sum(e, axis=0, keepdims=True)
    m = lax.dot_general(
        alpha.astype(jnp.bfloat16), xcb, (((0,), (0,)), ((), ())),
        preferred_element_type=jnp.float32,
    )

    mu = jnp.sum(muw * m, axis=1, keepdims=True) + mub
    lv = jnp.sum(vaw * m, axis=1, keepdims=True) + vab
    epsv = jnp.transpose(eps_ref[0])
    z = epsv * jnp.exp(0.5 * lv) + mu
    yv = jnp.sum(fw * jnp.transpose(z), axis=1, keepdims=True) + fb
    yt = jnp.transpose(yv)
    y_ref[0] = yt

    tgt = tgt_ref[0]
    bce_el = jnp.maximum(yt, 0.0) - yt * tgt + jnp.log1p(jnp.exp(-jnp.abs(yt)))
    bce_ref[0] = jnp.sum(bce_el).reshape(1, 1)
    kl_ref[0] = (-0.5 * jnp.sum(1.0 + lv - mu * mu - jnp.exp(lv))).reshape(1, 1)


def _prep_weights(params):
    ws = []
    for ch in params["channels"]:
        w0, b0 = _prep_conv(ch["base_w"], ch["base_b"])
        ws += [w0, b0]
        for blk in ch["blocks"]:
            w1, bb1 = _fold_bn(blk["w1"], blk["bn1_g"], blk["bn1_b"],
                               blk["bn1_m"], blk["bn1_v"])
            w2, bb2 = _fold_bn(blk["w2"], blk["bn2_g"], blk["bn2_b"],
                               blk["bn2_m"], blk["bn2_v"])
            wsc, bbs = _fold_bn(blk["ws"], blk["bns_g"], blk["bns_b"],
                                blk["bns_m"], blk["bns_v"])
            p1, pb1 = _prep_conv(w1, bb1)
            p2, pb2 = _prep_conv(w2, bb2)
            ps, pbs = _prep_conv(wsc, bbs)
            ws += [p1, pb1, p2, pb2, ps[0], pbs]
    nc = len(FILTER_SIZES)

    def padf(a):
        out = jnp.zeros((a.shape[0], FEATP), a.dtype)
        for c in range(nc):
            out = out.at[:, c * CP:c * CP + NFM].set(a[:, c * NFM:(c + 1) * NFM])
        return out

    ws.append(jnp.transpose(padf(params["U_w"])).astype(jnp.bfloat16))
    ws.append(padf(params["mu_w"]))
    ws.append(params["mu_b"].reshape(LATENT, 1))
    ws.append(padf(params["var_w"]))
    ws.append(params["var_b"].reshape(LATENT, 1))
    ws.append(params["final_w"])
    ws.append(params["final_b"].reshape(Y, 1))
    return ws


def _tc_forward(emb, target, eps, weights):
    nw = len(weights)

    def const_spec(a):
        return pl.BlockSpec(a.shape, lambda b: (0,) * a.ndim)

    in_specs = [
        pl.BlockSpec((1, L, D), lambda b: (b, 0, 0)),
        pl.BlockSpec((1, 1, Y), lambda b: (b, 0, 0)),
        pl.BlockSpec((1, 1, LATENT), lambda b: (b, 0, 0)),
    ] + [const_spec(w) for w in weights]
    out_specs = [
        pl.BlockSpec((1, 1, Y), lambda b: (b, 0, 0)),
        pl.BlockSpec((1, 1, 1), lambda b: (b, 0, 0)),
        pl.BlockSpec((1, 1, 1), lambda b: (b, 0, 0)),
    ]
    out_shape = [
        jax.ShapeDtypeStruct((B, 1, Y), jnp.float32),
        jax.ShapeDtypeStruct((B, 1, 1), jnp.float32),
        jax.ShapeDtypeStruct((B, 1, 1), jnp.float32),
    ]

    def body(*refs):
        _tc_kernel_body(refs, nweights=nw)

    y3, bce3, kl3 = pl.pallas_call(
        body,
        grid=(B,),
        in_specs=in_specs,
        out_specs=out_specs,
        out_shape=out_shape,
    )(emb.reshape(B, L, D), target.reshape(B, 1, Y),
      eps.reshape(B, 1, LATENT), *weights)
    return y3, bce3, kl3


def kernel(x, target, text_inputs, eps, params):
    del text_inputs
    emb = _sc_gather(params["embed"], x.reshape(-1))
    weights = _prep_weights(params)
    y3, bce3, kl3 = _tc_forward(emb, target, eps, weights)
    y = y3.reshape(B, Y)
    bce = jnp.sum(bce3) / (B * Y)
    kl = jnp.sum(kl3) / B
    return y, bce, kl
